# Initial kernel scaffold; baseline (speedup 1.0000x reference)
import jax
import jax.numpy as jnp
from jax.experimental import pallas as pl


def kernel(idx, table, targets):
    raise NotImplementedError("write your pallas kernel here")


# speedup vs baseline: 1.1038x; 1.1038x over previous
import jax
import jax.numpy as jnp
from jax.experimental import pallas as pl
from jax.experimental.pallas import tpu as pltpu

_TM = 256
_TR = 256


def _prep_kernel(table_ref, tbf_ref, lsec_ref):
    t = table_ref[...]
    tbf_ref[...] = t.astype(jnp.bfloat16)
    m = jnp.max(t, axis=-1, keepdims=True)
    lse = jnp.log(jnp.sum(jnp.exp(t - m), axis=-1, keepdims=True)) + m
    hi = lse.astype(jnp.bfloat16)
    hi32 = hi.astype(jnp.float32)
    lo32 = (lse - hi32).astype(jnp.bfloat16).astype(jnp.float32)
    col = jax.lax.broadcasted_iota(jnp.int32, lsec_ref.shape, 1)
    m0 = (col == 0).astype(jnp.float32)
    m1 = (col == 1).astype(jnp.float32)
    lsec_ref[...] = (hi32 * m0 + lo32 * m1).astype(jnp.bfloat16)


def _main_kernel(idx_ref, tgt_ref, tbf_ref, lsec_ref, logits_ref, nll_ref):
    idx = idx_ref[...]
    tgt = tgt_ref[...]
    tm = idx.shape[0]
    v = tbf_ref.shape[1]
    col = jax.lax.broadcasted_iota(jnp.int32, (tm, v), 1)
    oh = (col == idx).astype(jnp.bfloat16)
    acc = jnp.dot(oh, tbf_ref[...], preferred_element_type=jnp.float32)
    logits_ref[...] = acc
    acc2 = jnp.dot(oh, lsec_ref[...], preferred_element_type=jnp.float32)
    lse_tok = acc2[:, 0:1] + acc2[:, 1:2]
    tgt_logit = jnp.sum(jnp.where(col == tgt, acc, 0.0), axis=-1,
                        keepdims=True)
    nll_ref[...] = lse_tok - tgt_logit


def kernel(idx, table, targets):
    B, T = idx.shape
    V = table.shape[0]
    BT = B * T

    idx_flat = idx.reshape(BT, 1).astype(jnp.int32)
    tgt_flat = targets.reshape(BT, 1).astype(jnp.int32)
    table = table.astype(jnp.float32)

    n_prep = V // _TR
    tbf, lsec = pl.pallas_call(
        _prep_kernel,
        out_shape=(jax.ShapeDtypeStruct((V, V), jnp.bfloat16),
                   jax.ShapeDtypeStruct((V, 128), jnp.bfloat16)),
        grid=(n_prep,),
        in_specs=[pl.BlockSpec((_TR, V), lambda i: (i, 0))],
        out_specs=(pl.BlockSpec((_TR, V), lambda i: (i, 0)),
                   pl.BlockSpec((_TR, 128), lambda i: (i, 0))),
        compiler_params=pltpu.CompilerParams(
            dimension_semantics=("parallel",),
            vmem_limit_bytes=int(64 << 20)),
        cost_estimate=pl.CostEstimate(
            flops=3 * V * V,
            transcendentals=V * V,
            bytes_accessed=V * V * 4 + V * V * 2 + V * 128 * 2),
    )(table)

    n_blocks = BT // _TM
    cost = pl.CostEstimate(
        flops=2 * BT * V * (V + 128),
        transcendentals=0,
        bytes_accessed=(2 * BT * 4 + V * V * 2 + V * 128 * 2
                        + BT * V * 4 + BT * 4))
    logits, nll = pl.pallas_call(
        _main_kernel,
        out_shape=(jax.ShapeDtypeStruct((BT, V), jnp.float32),
                   jax.ShapeDtypeStruct((BT, 1), jnp.float32)),
        grid=(n_blocks,),
        in_specs=[pl.BlockSpec((_TM, 1), lambda i: (i, 0)),
                  pl.BlockSpec((_TM, 1), lambda i: (i, 0)),
                  pl.BlockSpec((V, V), lambda i: (0, 0)),
                  pl.BlockSpec((V, 128), lambda i: (0, 0))],
        out_specs=(pl.BlockSpec((_TM, V), lambda i: (i, 0)),
                   pl.BlockSpec((_TM, 1), lambda i: (i, 0))),
        compiler_params=pltpu.CompilerParams(
            dimension_semantics=("parallel",),
            vmem_limit_bytes=int(56 << 20)),
        cost_estimate=cost,
    )(idx_flat, tgt_flat, tbf, lsec)

    loss = jnp.sum(nll[:, 0]) / BT
    return logits, loss
